# Initial kernel scaffold; baseline (speedup 1.0000x reference)
import jax
import jax.numpy as jnp
from jax.experimental import pallas as pl


def kernel(frame, attributes, mask):
    raise NotImplementedError("write your pallas kernel here")


# speedup vs baseline: 8.7834x; 8.7834x over previous
import functools

import jax
import jax.numpy as jnp
from jax.experimental import pallas as pl
from jax.experimental.pallas import tpu as pltpu

_B, _N, _D, _K = 4, 2048, 128, 16
_BN = 256


def _topk_gather_body(ctr_ref, aux_ref, tab_ref, coord_ref, attr_ref):
    x = ctr_ref[0]
    aux = aux_ref[0]

    dist = aux[3:4, :]
    for c in range(3):
        d = x[:, c:c + 1] - aux[c:c + 1, :]
        dist = dist + d * d

    lanes = jax.lax.broadcasted_iota(jnp.int32, (_BN, _N), 1)
    lanes128 = jax.lax.broadcasted_iota(jnp.int32, (_BN, 128), 1)
    tab = tab_ref[0]
    cacc = jnp.zeros((_BN, 128), jnp.float32)

    for k in range(_K):
        m = jnp.min(dist, axis=1, keepdims=True)
        eq = dist == m
        idx = jnp.min(jnp.where(eq, lanes, _N), axis=1, keepdims=True)
        oh = lanes == idx
        dist = jnp.where(oh, jnp.inf, dist)
        g = jax.lax.dot(oh.astype(jnp.float32), tab,
                        preferred_element_type=jnp.float32)
        attr_ref[0, :, k, :] = g[:, :_D]
        for cp in range(3):
            e = jnp.zeros((_BN, 1), jnp.float32)
            for c in range(3):
                delta = g[:, _D + c:_D + c + 1] - x[:, c:c + 1]
                e = e + x[:, 3 + 3 * cp + c:4 + 3 * cp + c] * delta
            cacc = jnp.where(lanes128 == (3 * k + cp), e, cacc)

    coord_ref[0] = cacc


@jax.jit
def kernel(frame, attributes, mask):
    B, N, D, K = _B, _N, _D, _K
    center = frame[:, :, 0]
    rot = frame[:, :, 1:4].reshape(B, N, 9)
    ctr_pad = jnp.zeros((B, N, 128), jnp.float32)
    ctr_pad = ctr_pad.at[:, :, 0:3].set(center).at[:, :, 3:12].set(rot)

    pen = (1.0 - mask[0][:, :, 1]) * 2000.0
    aux = jnp.zeros((B, 8, N), jnp.float32)
    aux = aux.at[:, 0:3, :].set(center.transpose(0, 2, 1))
    aux = aux.at[:, 3, :].set(pen)

    tab = jnp.zeros((B, N, 256), jnp.float32)
    tab = tab.at[:, :, :D].set(attributes).at[:, :, D:D + 3].set(center)

    nblk = N // _BN
    grid = (B, nblk)
    coords_pad, attrs = pl.pallas_call(
        _topk_gather_body,
        grid=grid,
        in_specs=[
            pl.BlockSpec((1, _BN, 128), lambda b, i: (b, i, 0)),
            pl.BlockSpec((1, 8, N), lambda b, i: (b, 0, 0)),
            pl.BlockSpec((1, N, 256), lambda b, i: (b, 0, 0)),
        ],
        out_specs=[
            pl.BlockSpec((1, _BN, 128), lambda b, i: (b, i, 0)),
            pl.BlockSpec((1, _BN, K, D), lambda b, i: (b, i, 0, 0)),
        ],
        out_shape=[
            jax.ShapeDtypeStruct((B, N, 128), jnp.float32),
            jax.ShapeDtypeStruct((B, N, K, D), jnp.float32),
        ],
    )(ctr_pad, aux, tab)

    coords = coords_pad[:, :, :3 * K].reshape(B, N, K, 3)
    return (coords, attrs)
